# baseline (speedup 1.0000x reference)
import functools

import jax
import jax.numpy as jnp
from jax.experimental import pallas as pl
from jax.experimental.pallas import tpu as pltpu

TOPK = 10
CHUNK = 128
BLK = 16384
J = 16
NEG = -3.0e38


def _qnorm(q):
    n = jnp.sqrt(jnp.sum(q * q, axis=1, keepdims=True))
    return q / (n + 1e-8)


def _sims_exact(qn, kb):
    ss = jnp.sum(kb * kb, axis=1, keepdims=True)
    kn = kb / (jnp.sqrt(ss) + 1e-8)
    return jax.lax.dot_general(
        qn, kn, (((1,), (1,)), ((), ())),
        preferred_element_type=jnp.float32)


def _phase1_kernel(q_ref, k_ref, bmax_ref):
    qn = _qnorm(q_ref[...])
    kb = k_ref[...]
    ss = jax.lax.dot_general(
        jnp.ones((1, kb.shape[1]), jnp.float32), kb * kb,
        (((1,), (1,)), ((), ())), preferred_element_type=jnp.float32,
        precision=jax.lax.Precision.HIGHEST)
    scale = 1.0 / (jnp.sqrt(ss) + 1e-8)
    sim = jax.lax.dot_general(
        qn, kb, (((1,), (1,)), ((), ())),
        preferred_element_type=jnp.float32) * scale
    sim3 = sim.reshape(sim.shape[0], BLK // CHUNK, CHUNK)
    bmax_ref[...] = jnp.max(sim3, axis=2)


def _phaseB_kernel(bmax_ref, q_ref, t_ref, ids_ref, qn_ref, *, gmain, tail_len):
    bm = bmax_ref[...]
    qn = _qnorm(q_ref[...])
    qn_ref[...] = qn
    simt = _sims_exact(qn, t_ref[...])
    tcol = jax.lax.broadcasted_iota(jnp.int32, (1, simt.shape[1]), 1)
    simt = jnp.where(tcol < tail_len, simt, NEG)
    tg = simt.shape[1] // CHUNK
    bt = jnp.max(simt.reshape(simt.shape[0], tg, CHUNK), axis=2)

    gid_m = jax.lax.broadcasted_iota(jnp.int32, bm.shape, 1)
    gid_t = jax.lax.broadcasted_iota(jnp.int32, bt.shape, 1) + gmain
    lane = jax.lax.broadcasted_iota(jnp.int32, ids_ref.shape, 1)
    ids = jnp.zeros(ids_ref.shape, jnp.int32)
    for j in range(J):
        m = jnp.maximum(jnp.max(bm, axis=1, keepdims=True),
                        jnp.max(bt, axis=1, keepdims=True))
        sel = jnp.minimum(
            jnp.min(jnp.where(bm == m, gid_m, 2**30), axis=1, keepdims=True),
            jnp.min(jnp.where(bt == m, gid_t, 2**30), axis=1, keepdims=True))
        ids = jnp.where(lane == j, sel, ids)
        bm = jnp.where(gid_m == sel, NEG, bm)
        bt = jnp.where(gid_t == sel, NEG, bt)
    ids_ref[...] = ids


def _phase2_kernel(idr_ref, idm_ref, idt_ref, qn_ref, *rest, gmain, n_real):
    km = rest[:J]
    kt = rest[J:2 * J]
    so_ref, io_ref = rest[2 * J], rest[2 * J + 1]
    t = pl.program_id(0)
    qn = qn_ref[...]
    row = jax.lax.broadcasted_iota(jnp.int32, qn.shape, 0)
    qsel = jnp.max(jnp.where(row == t, qn, NEG), axis=0,
                   keepdims=True)
    sims = []
    cols = []
    ci = jax.lax.broadcasted_iota(jnp.int32, (1, CHUNK), 1)
    for j in range(J):
        idj = idr_ref[t * J + j]
        kb = jnp.where(idj >= gmain, kt[j][...], km[j][...])
        col = ci + idj * CHUNK
        sims.append(jnp.where(col < n_real, _sims_exact(qsel, kb), NEG))
        cols.append(col)
    s = jnp.concatenate(sims, axis=1)
    idx = jnp.concatenate(cols, axis=1)
    lane = jax.lax.broadcasted_iota(jnp.int32, (1, CHUNK), 1)
    so = jnp.full((1, CHUNK), NEG, jnp.float32)
    io = jnp.zeros((1, CHUNK), jnp.int32)
    for r in range(TOPK):
        m = jnp.max(s, axis=1, keepdims=True)
        mi = jnp.min(jnp.where(s == m, idx, 2**30), axis=1,
                     keepdims=True)
        so = jnp.where(lane == r, m, so)
        io = jnp.where(lane == r, mi, io)
        s = jnp.where(idx == mi, NEG, s)
    so_ref[...] = so.reshape(1, 1, CHUNK)
    io_ref[...] = io.reshape(1, 1, CHUNK)


def kernel(queries, keys):
    q, d = queries.shape
    n, _ = keys.shape
    nmain = (n // BLK) * BLK
    nblk = nmain // BLK
    gmain = nmain // CHUNK
    tail_len = n - nmain
    tg = max(1, -(-tail_len // CHUNK))
    tpad = jnp.pad(keys[nmain:], ((0, tg * CHUNK - tail_len), (0, 0)))

    bmax = pl.pallas_call(
        _phase1_kernel,
        grid=(nblk,),
        in_specs=[
            pl.BlockSpec((q, d), lambda i: (0, 0)),
            pl.BlockSpec((BLK, d), lambda i: (i, 0)),
        ],
        out_specs=pl.BlockSpec((q, BLK // CHUNK), lambda i: (0, i)),
        out_shape=jax.ShapeDtypeStruct((q, gmain), jnp.float32),
    )(queries, keys)

    ids_mat, qn = pl.pallas_call(
        functools.partial(_phaseB_kernel, gmain=gmain, tail_len=tail_len),
        out_shape=[jax.ShapeDtypeStruct((q, 128), jnp.int32),
                   jax.ShapeDtypeStruct((q, d), jnp.float32)],
    )(bmax, queries, tpad)
    ids_raw = ids_mat[:, :J].reshape(-1)
    ids_main = jnp.minimum(ids_raw, gmain - 1)
    ids_tail = jnp.clip(ids_raw - gmain, 0, tg - 1)

    main_spec = [
        pl.BlockSpec(
            (CHUNK, d),
            functools.partial(
                lambda t, idr, idm, idt, jj: (idm[t * J + jj], 0), jj=j))
        for j in range(J)
    ]
    tail_spec = [
        pl.BlockSpec(
            (CHUNK, d),
            functools.partial(
                lambda t, idr, idm, idt, jj: (idt[t * J + jj], 0), jj=j))
        for j in range(J)
    ]
    so3, io3 = pl.pallas_call(
        functools.partial(_phase2_kernel, gmain=gmain, n_real=n),
        grid_spec=pltpu.PrefetchScalarGridSpec(
            num_scalar_prefetch=3,
            grid=(q,),
            in_specs=[pl.BlockSpec((q, d), lambda t, idr, idm, idt: (0, 0))]
                     + main_spec + tail_spec,
            out_specs=[
                pl.BlockSpec((1, 1, CHUNK), lambda t, idr, idm, idt: (t, 0, 0)),
                pl.BlockSpec((1, 1, CHUNK), lambda t, idr, idm, idt: (t, 0, 0)),
            ],
        ),
        out_shape=[
            jax.ShapeDtypeStruct((q, 1, CHUNK), jnp.float32),
            jax.ShapeDtypeStruct((q, 1, CHUNK), jnp.int32),
        ],
    )(ids_raw, ids_main, ids_tail, qn, *([keys] * J), *([tpad] * J))

    return so3.reshape(q, CHUNK)[:, :TOPK], io3.reshape(q, CHUNK)[:, :TOPK]
